# Initial kernel scaffold; baseline (speedup 1.0000x reference)
import jax
import jax.numpy as jnp
from jax.experimental import pallas as pl


def kernel(z, embedding_weight, proj_W, proj_b, ema_usage):
    raise NotImplementedError("write your pallas kernel here")


# speedup vs baseline: 1.1061x; 1.1061x over previous
import functools

import jax
import jax.numpy as jnp
from jax import lax
from jax.experimental import pallas as pl
from jax.experimental.pallas import tpu as pltpu
from jax.experimental.pallas import tpu_sc as plsc

K = 8192
D = 256
N = 9216
BETA = 0.25
EMA_DECAY = 0.99

KB0 = 512
NB = 256
KC = 2048
NBLK = N // NB

NC, NS = 2, 16
NW = NC * NS
BPW = N // NW
GW = 96
NGC = BPW // GW


def _k0_body(e_ref, w_ref, b_ref, qc_ref, qct_ref, cnb_ref):
    e = e_ref[...]
    w = w_ref[...]
    qc = lax.dot_general(e, w, (((1,), (1,)), ((), ())),
                         preferred_element_type=jnp.float32)
    qc = qc + b_ref[...]
    qc_ref[...] = qc
    qct = qc.T
    qct_ref[...] = qct
    cn = jnp.sum(qct * qct, axis=0, keepdims=True)
    cnb_ref[...] = jnp.broadcast_to(cn, (8, KB0))


def _project(emb, w, b2d):
    return pl.pallas_call(
        _k0_body,
        grid=(K // KB0,),
        in_specs=[
            pl.BlockSpec((KB0, D), lambda i: (i, 0)),
            pl.BlockSpec((D, D), lambda i: (0, 0)),
            pl.BlockSpec((1, D), lambda i: (0, 0)),
        ],
        out_specs=[
            pl.BlockSpec((KB0, D), lambda i: (i, 0)),
            pl.BlockSpec((D, KB0), lambda i: (0, i)),
            pl.BlockSpec((8, KB0), lambda i: (0, i)),
        ],
        out_shape=[
            jax.ShapeDtypeStruct((K, D), jnp.float32),
            jax.ShapeDtypeStruct((D, K), jnp.float32),
            jax.ShapeDtypeStruct((8, K), jnp.float32),
        ],
    )(emb, w, b2d)


def _k1_body(z_ref, qct_ref, cnb_ref, idx_ref, mind_ref, cnt_ref):
    z = z_ref[...]
    zn2 = jnp.sum(z * z, axis=1, keepdims=True)
    best_min = None
    best_idx = None
    for c in range(K // KC):
        qct_c = qct_ref[:, c * KC:(c + 1) * KC]
        cn_c = cnb_ref[0:1, c * KC:(c + 1) * KC]
        dot = jnp.dot(z, qct_c, preferred_element_type=jnp.float32)
        d = (zn2 + cn_c) - 2.0 * dot
        lmin = jnp.min(d, axis=1, keepdims=True)
        col = lax.broadcasted_iota(jnp.int32, (NB, KC), 1) + c * KC
        lidx = jnp.min(jnp.where(d == lmin, col, jnp.int32(2 ** 30)),
                       axis=1, keepdims=True)
        if best_min is None:
            best_min, best_idx = lmin, lidx
        else:
            upd = lmin < best_min
            best_min = jnp.where(upd, lmin, best_min)
            best_idx = jnp.where(upd, lidx, best_idx)
    idx_ref[...] = best_idx
    mind_ref[...] = best_min
    parts = []
    for c in range(K // KC):
        col = lax.broadcasted_iota(jnp.int32, (NB, KC), 1) + c * KC
        eq = (col == best_idx).astype(jnp.float32)
        parts.append(jnp.sum(eq, axis=0))
    cnt_ref[...] = jnp.concatenate(parts).reshape(1, 1, K)


def _argmin_dist(z2d, qct, cnb):
    return pl.pallas_call(
        _k1_body,
        grid=(NBLK,),
        in_specs=[
            pl.BlockSpec((NB, D), lambda i: (i, 0)),
            pl.BlockSpec((D, K), lambda i: (0, 0)),
            pl.BlockSpec((8, K), lambda i: (0, 0)),
        ],
        out_specs=[
            pl.BlockSpec((NB, 1), lambda i: (i, 0)),
            pl.BlockSpec((NB, 1), lambda i: (i, 0)),
            pl.BlockSpec((1, 1, K), lambda i: (i, 0, 0)),
        ],
        out_shape=[
            jax.ShapeDtypeStruct((N, 1), jnp.int32),
            jax.ShapeDtypeStruct((N, 1), jnp.float32),
            jax.ShapeDtypeStruct((NBLK, 1, K), jnp.float32),
        ],
    )(z2d, qct, cnb)


def _sc_body(qc_hbm, idx_hbm, zq_hbm, idx_v0, idx_v1, idx_v2, rows_v, gsem):
    c = lax.axis_index("c")
    s = lax.axis_index("s")
    wid = s * NC + c
    idx_vs = (idx_v0, idx_v1, idx_v2)
    handles = []
    for j in range(NGC):
        pltpu.sync_copy(idx_hbm.at[pl.ds(wid * BPW + j * GW, GW)], idx_vs[j])
        handles.append(
            pltpu.async_copy(qc_hbm.at[idx_vs[j]],
                             rows_v.at[pl.ds(j * GW, GW)], gsem))
    for h in handles:
        h.wait()
    pltpu.sync_copy(rows_v, zq_hbm.at[pl.ds(wid * BPW, BPW)])


def _sc_gather(qc, idx1d):
    mesh = plsc.VectorSubcoreMesh(core_axis_name="c", subcore_axis_name="s")
    kern = pl.kernel(
        _sc_body,
        out_type=jax.ShapeDtypeStruct((N, D), jnp.float32),
        mesh=mesh,
        scratch_types=[
            pltpu.VMEM((GW,), jnp.int32),
            pltpu.VMEM((GW,), jnp.int32),
            pltpu.VMEM((GW,), jnp.int32),
            pltpu.VMEM((BPW, D), jnp.float32),
            pltpu.SemaphoreType.DMA,
        ],
    )
    return kern(qc, idx1d)


def _k2_body(mind_ref, cnts_ref, ema_ref, out_ref):
    counts = jnp.sum(cnts_ref[...], axis=0)
    e_mean = counts / jnp.float32(N)
    perp = jnp.exp(-jnp.sum(e_mean * jnp.log(e_mean + 1e-10)))
    new_ema = ema_ref[...] * EMA_DECAY + (1.0 - EMA_DECAY) * e_mean
    ent = -jnp.sum(new_ema * jnp.log(new_ema + 1e-10))
    m = jnp.sum(mind_ref[...]) / jnp.float32(N * D)
    loss = BETA * m + m
    sub = lax.broadcasted_iota(jnp.int32, (8, 128), 0)
    lane = lax.broadcasted_iota(jnp.int32, (8, 128), 1)
    r0 = sub == 0
    out_ref[...] = jnp.where(
        r0 & (lane == 0), loss,
        jnp.where(r0 & (lane == 1), perp,
                  jnp.where(r0 & (lane == 2), ent, 0.0))).astype(jnp.float32)


def _finalize(mind, pcounts, ema1):
    return pl.pallas_call(
        _k2_body,
        out_shape=jax.ShapeDtypeStruct((8, 128), jnp.float32),
    )(mind, pcounts.reshape(NBLK, K), ema1)


@functools.partial(jax.jit)
def kernel(z, embedding_weight, proj_W, proj_b, ema_usage):
    z2d = z.reshape(N, D)
    b2d = proj_b.reshape(1, D)
    qc, qct, cnb = _project(embedding_weight, proj_W, b2d)
    idx, mind, pcounts = _argmin_dist(z2d, qct, cnb)
    zq = _sc_gather(qc, idx.reshape(N))
    out2 = _finalize(mind, pcounts, ema_usage.reshape(1, K))
    z_q_st = zq.reshape(z.shape)
    return (z_q_st, out2[0, 0], out2[0, 1], out2[0, 2])


# speedup vs baseline: 1.3428x; 1.2140x over previous
import functools

import jax
import jax.numpy as jnp
from jax import lax
from jax.experimental import pallas as pl
from jax.experimental.pallas import tpu as pltpu
from jax.experimental.pallas import tpu_sc as plsc

K = 8192
D = 256
N = 9216
BETA = 0.25
EMA_DECAY = 0.99

KB0 = 512
NB = 256
KC = 2048
NBLK = N // NB

NC, NS = 2, 16
NW = NC * NS
BPW = N // NW
GW = 96
NGC = BPW // GW


def _k0_body(e_ref, w_ref, b_ref, qc_ref, qct_ref, cnb_ref):
    e = e_ref[...]
    w = w_ref[...]
    qc = lax.dot_general(e, w, (((1,), (1,)), ((), ())),
                         preferred_element_type=jnp.float32)
    qc = qc + b_ref[...]
    qc_ref[...] = qc
    qct = qc.T
    qct_ref[...] = qct
    cn = jnp.sum(qct * qct, axis=0, keepdims=True)
    cnb_ref[...] = jnp.broadcast_to(cn, (8, KB0))


def _project(emb, w, b2d):
    return pl.pallas_call(
        _k0_body,
        grid=(K // KB0,),
        in_specs=[
            pl.BlockSpec((KB0, D), lambda i: (i, 0)),
            pl.BlockSpec((D, D), lambda i: (0, 0)),
            pl.BlockSpec((1, D), lambda i: (0, 0)),
        ],
        out_specs=[
            pl.BlockSpec((KB0, D), lambda i: (i, 0)),
            pl.BlockSpec((D, KB0), lambda i: (0, i)),
            pl.BlockSpec((8, KB0), lambda i: (0, i)),
        ],
        out_shape=[
            jax.ShapeDtypeStruct((K, D), jnp.float32),
            jax.ShapeDtypeStruct((D, K), jnp.float32),
            jax.ShapeDtypeStruct((8, K), jnp.float32),
        ],
    )(emb, w, b2d)


def _pair_tree(d, g, base):
    w = d.shape[1]
    lvl = 0
    while w > 128:
        half = w // 128 // 2
        nd, ng = [], []
        for j in range(half):
            a = d[:, (2 * j) * 128:(2 * j + 1) * 128]
            b = d[:, (2 * j + 1) * 128:(2 * j + 2) * 128]
            cmp = b < a
            nd.append(jnp.minimum(a, b))
            if g is None:
                ng.append(jnp.where(cmp, jnp.int32(base + 2 * j + 1),
                                    jnp.int32(base + 2 * j)))
            else:
                ga = g[:, (2 * j) * 128:(2 * j + 1) * 128]
                gb = g[:, (2 * j + 1) * 128:(2 * j + 2) * 128]
                ng.append(jnp.where(cmp, gb, ga))
        d = jnp.concatenate(nd, axis=1)
        g = jnp.concatenate(ng, axis=1)
        w //= 2
        lvl += 1
    return d, g


def _k1_body(z_ref, qct_ref, cnb_ref, idx_ref, mind_ref, cnt_ref, d_scr):
    z = z_ref[...]
    zn2 = jnp.sum(z * z, axis=1, keepdims=True)
    zm2 = -2.0 * z
    run_d = None
    run_g = None
    for c in range(K // KC):
        qct_c = qct_ref[:, c * KC:(c + 1) * KC]
        cn_c = cnb_ref[0:1, c * KC:(c + 1) * KC]
        dotm2 = jnp.dot(zm2, qct_c, preferred_element_type=jnp.float32)
        d = (zn2 + cn_c) + dotm2
        d_scr[:, c * KC:(c + 1) * KC] = d
        cd, cg = _pair_tree(d, None, c * (KC // 128))
        if run_d is None:
            run_d, run_g = cd, cg
        else:
            upd = cd < run_d
            run_d = jnp.where(upd, cd, run_d)
            run_g = jnp.where(upd, cg, run_g)
    lane = lax.broadcasted_iota(jnp.int32, (NB, 128), 1)
    col = run_g * 128 + lane
    best_min = jnp.min(run_d, axis=1, keepdims=True)
    best_idx = jnp.min(jnp.where(run_d == best_min, col, jnp.int32(2 ** 30)),
                       axis=1, keepdims=True)
    idx_ref[...] = best_idx
    mind_ref[...] = best_min
    eq = (d_scr[...] == best_min)
    cnt_ref[...] = jnp.sum(jnp.where(eq, 1.0, 0.0), axis=0,
                           dtype=jnp.float32).reshape(1, 1, K)


def _argmin_dist(z2d, qct, cnb):
    return pl.pallas_call(
        _k1_body,
        grid=(NBLK,),
        in_specs=[
            pl.BlockSpec((NB, D), lambda i: (i, 0)),
            pl.BlockSpec((D, K), lambda i: (0, 0)),
            pl.BlockSpec((8, K), lambda i: (0, 0)),
        ],
        out_specs=[
            pl.BlockSpec((NB, 1), lambda i: (i, 0)),
            pl.BlockSpec((NB, 1), lambda i: (i, 0)),
            pl.BlockSpec((1, 1, K), lambda i: (i, 0, 0)),
        ],
        out_shape=[
            jax.ShapeDtypeStruct((N, 1), jnp.int32),
            jax.ShapeDtypeStruct((N, 1), jnp.float32),
            jax.ShapeDtypeStruct((NBLK, 1, K), jnp.float32),
        ],
        scratch_shapes=[pltpu.VMEM((NB, K), jnp.float32)],
    )(z2d, qct, cnb)


def _sc_body(qc_hbm, idx_hbm, zq_hbm, idx_v0, idx_v1, idx_v2, rows_v, gsem):
    c = lax.axis_index("c")
    s = lax.axis_index("s")
    wid = s * NC + c
    idx_vs = (idx_v0, idx_v1, idx_v2)
    handles = []
    for j in range(NGC):
        pltpu.sync_copy(idx_hbm.at[pl.ds(wid * BPW + j * GW, GW)], idx_vs[j])
        handles.append(
            pltpu.async_copy(qc_hbm.at[idx_vs[j]],
                             rows_v.at[pl.ds(j * GW, GW)], gsem))
    for h in handles:
        h.wait()
    pltpu.sync_copy(rows_v, zq_hbm.at[pl.ds(wid * BPW, BPW)])


def _sc_gather(qc, idx1d):
    mesh = plsc.VectorSubcoreMesh(core_axis_name="c", subcore_axis_name="s")
    kern = pl.kernel(
        _sc_body,
        out_type=jax.ShapeDtypeStruct((N, D), jnp.float32),
        mesh=mesh,
        scratch_types=[
            pltpu.VMEM((GW,), jnp.int32),
            pltpu.VMEM((GW,), jnp.int32),
            pltpu.VMEM((GW,), jnp.int32),
            pltpu.VMEM((BPW, D), jnp.float32),
            pltpu.SemaphoreType.DMA,
        ],
    )
    return kern(qc, idx1d)


def _k2_body(mind_ref, cnts_ref, ema_ref, out_ref):
    counts = jnp.sum(cnts_ref[...], axis=0)
    e_mean = counts / jnp.float32(N)
    perp = jnp.exp(-jnp.sum(e_mean * jnp.log(e_mean + 1e-10)))
    new_ema = ema_ref[...] * EMA_DECAY + (1.0 - EMA_DECAY) * e_mean
    ent = -jnp.sum(new_ema * jnp.log(new_ema + 1e-10))
    m = jnp.sum(mind_ref[...]) / jnp.float32(N * D)
    loss = BETA * m + m
    sub = lax.broadcasted_iota(jnp.int32, (8, 128), 0)
    lane = lax.broadcasted_iota(jnp.int32, (8, 128), 1)
    r0 = sub == 0
    out_ref[...] = jnp.where(
        r0 & (lane == 0), loss,
        jnp.where(r0 & (lane == 1), perp,
                  jnp.where(r0 & (lane == 2), ent, 0.0))).astype(jnp.float32)


def _finalize(mind, pcounts, ema1):
    return pl.pallas_call(
        _k2_body,
        out_shape=jax.ShapeDtypeStruct((8, 128), jnp.float32),
    )(mind, pcounts.reshape(NBLK, K), ema1)


@functools.partial(jax.jit)
def kernel(z, embedding_weight, proj_W, proj_b, ema_usage):
    z2d = z.reshape(N, D)
    b2d = proj_b.reshape(1, D)
    qc, qct, cnb = _project(embedding_weight, proj_W, b2d)
    idx, mind, pcounts = _argmin_dist(z2d, qct, cnb)
    zq = _sc_gather(qc, idx.reshape(N))
    out2 = _finalize(mind, pcounts, ema_usage.reshape(1, K))
    z_q_st = zq.reshape(z.shape)
    return (z_q_st, out2[0, 0], out2[0, 1], out2[0, 2])


# speedup vs baseline: 1.6095x; 1.1986x over previous
import functools

import jax
import jax.numpy as jnp
from jax import lax
from jax.experimental import pallas as pl
from jax.experimental.pallas import tpu as pltpu
from jax.experimental.pallas import tpu_sc as plsc

K = 8192
D = 256
N = 9216
BETA = 0.25
EMA_DECAY = 0.99

KB0 = 512
NB = 256
KC = 2048
NBLK = N // NB

NC, NS = 2, 16
NW = NC * NS
BPW = N // NW
GW = 96
NGC = BPW // GW


def _k0_body(e_ref, w_ref, b_ref, qc_ref, qct_ref, cnb_ref):
    e = e_ref[...]
    w = w_ref[...]
    qc = lax.dot_general(e, w, (((1,), (1,)), ((), ())),
                         preferred_element_type=jnp.float32)
    qc = qc + b_ref[...]
    qc_ref[...] = qc
    qct = qc.T
    qct_ref[...] = qct
    cn = jnp.sum(qct * qct, axis=0, keepdims=True)
    cnb_ref[...] = jnp.broadcast_to(cn, (8, KB0))


def _project(emb, w, b2d):
    return pl.pallas_call(
        _k0_body,
        grid=(K // KB0,),
        in_specs=[
            pl.BlockSpec((KB0, D), lambda i: (i, 0)),
            pl.BlockSpec((D, D), lambda i: (0, 0)),
            pl.BlockSpec((1, D), lambda i: (0, 0)),
        ],
        out_specs=[
            pl.BlockSpec((KB0, D), lambda i: (i, 0)),
            pl.BlockSpec((D, KB0), lambda i: (0, i)),
            pl.BlockSpec((8, KB0), lambda i: (0, i)),
        ],
        out_shape=[
            jax.ShapeDtypeStruct((K, D), jnp.float32),
            jax.ShapeDtypeStruct((D, K), jnp.float32),
            jax.ShapeDtypeStruct((8, K), jnp.float32),
        ],
    )(emb, w, b2d)


def _pair_tree(d, g, base):
    w = d.shape[1]
    lvl = 0
    while w > 128:
        half = w // 128 // 2
        nd, ng = [], []
        for j in range(half):
            a = d[:, (2 * j) * 128:(2 * j + 1) * 128]
            b = d[:, (2 * j + 1) * 128:(2 * j + 2) * 128]
            cmp = b < a
            nd.append(jnp.minimum(a, b))
            if g is None:
                ng.append(jnp.where(cmp, jnp.int32(base + 2 * j + 1),
                                    jnp.int32(base + 2 * j)))
            else:
                ga = g[:, (2 * j) * 128:(2 * j + 1) * 128]
                gb = g[:, (2 * j + 1) * 128:(2 * j + 2) * 128]
                ng.append(jnp.where(cmp, gb, ga))
        d = jnp.concatenate(nd, axis=1)
        g = jnp.concatenate(ng, axis=1)
        w //= 2
        lvl += 1
    return d, g


def _k1_body(z_ref, qct_ref, cnb_ref, idx_ref, mind_ref):
    z = z_ref[...]
    zn2 = jnp.sum(z * z, axis=1, keepdims=True)
    zm2 = -2.0 * z
    run_d = None
    run_g = None
    for c in range(K // KC):
        qct_c = qct_ref[:, c * KC:(c + 1) * KC]
        cn_c = cnb_ref[0:1, c * KC:(c + 1) * KC]
        dotm2 = jnp.dot(zm2, qct_c, preferred_element_type=jnp.float32)
        d = (zn2 + cn_c) + dotm2
        cd, cg = _pair_tree(d, None, c * (KC // 128))
        if run_d is None:
            run_d, run_g = cd, cg
        else:
            upd = cd < run_d
            run_d = jnp.where(upd, cd, run_d)
            run_g = jnp.where(upd, cg, run_g)
    lane = lax.broadcasted_iota(jnp.int32, (NB, 128), 1)
    col = run_g * 128 + lane
    best_min = jnp.min(run_d, axis=1, keepdims=True)
    best_idx = jnp.min(jnp.where(run_d == best_min, col, jnp.int32(2 ** 30)),
                       axis=1, keepdims=True)
    idx_ref[...] = best_idx
    mind_ref[...] = best_min


def _argmin_dist(z2d, qct, cnb):
    return pl.pallas_call(
        _k1_body,
        grid=(NBLK,),
        in_specs=[
            pl.BlockSpec((NB, D), lambda i: (i, 0)),
            pl.BlockSpec((D, K), lambda i: (0, 0)),
            pl.BlockSpec((8, K), lambda i: (0, 0)),
        ],
        out_specs=[
            pl.BlockSpec((NB, 1), lambda i: (i, 0)),
            pl.BlockSpec((NB, 1), lambda i: (i, 0)),
        ],
        out_shape=[
            jax.ShapeDtypeStruct((N, 1), jnp.int32),
            jax.ShapeDtypeStruct((N, 1), jnp.float32),
        ],
    )(z2d, qct, cnb)


def _sc_body(qc_hbm, idx_hbm, ones_hbm, zeros_hbm, zq_hbm, hists_hbm,
             idx_v0, idx_v1, idx_v2, rows_v, ones_v, off_v, hist_v,
             shared_hist, gsem):
    c = lax.axis_index("c")
    s = lax.axis_index("s")
    wid = s * NC + c
    idx_vs = (idx_v0, idx_v1, idx_v2)
    handles = []
    for j in range(NGC):
        pltpu.sync_copy(idx_hbm.at[pl.ds(wid * BPW + j * GW, GW)], idx_vs[j])
        handles.append(
            pltpu.async_copy(qc_hbm.at[idx_vs[j]],
                             rows_v.at[pl.ds(j * GW, GW)], gsem))
    pltpu.sync_copy(zeros_hbm, hist_v)
    pltpu.sync_copy(hist_v, shared_hist.at[pl.ds(s * K, K)])
    pltpu.sync_copy(ones_hbm, ones_v)
    soff = s * K
    for j in range(NGC):
        for t in range(GW // 16):
            sl = pl.ds(t * 16, 16)
            off_v[sl] = idx_vs[j][sl] + soff
        pltpu.sync_copy(ones_v, shared_hist.at[off_v], add=True)
    pltpu.sync_copy(shared_hist.at[pl.ds(s * K, K)], hist_v)
    pltpu.sync_copy(hist_v, hists_hbm.at[pl.ds(wid * K, K)])
    for h in handles:
        h.wait()
    pltpu.sync_copy(rows_v, zq_hbm.at[pl.ds(wid * BPW, BPW)])


def _sc_gather_hist(qc, idx1d, ones, zeros):
    mesh = plsc.VectorSubcoreMesh(core_axis_name="c", subcore_axis_name="s")
    kern = pl.kernel(
        _sc_body,
        out_type=[
            jax.ShapeDtypeStruct((N, D), jnp.float32),
            jax.ShapeDtypeStruct((NW * K,), jnp.float32),
        ],
        mesh=mesh,
        scratch_types=[
            pltpu.VMEM((GW,), jnp.int32),
            pltpu.VMEM((GW,), jnp.int32),
            pltpu.VMEM((GW,), jnp.int32),
            pltpu.VMEM((BPW, D), jnp.float32),
            pltpu.VMEM((GW,), jnp.float32),
            pltpu.VMEM((GW,), jnp.int32),
            pltpu.VMEM((K,), jnp.float32),
            pltpu.VMEM_SHARED((NS * K,), jnp.float32),
            pltpu.SemaphoreType.DMA,
        ],
    )
    return kern(qc, idx1d, ones, zeros)


def _k2_body(mind_ref, cnts_ref, ema_ref, out_ref):
    counts = jnp.sum(cnts_ref[...], axis=0)
    e_mean = counts / jnp.float32(N)
    perp = jnp.exp(-jnp.sum(e_mean * jnp.log(e_mean + 1e-10)))
    new_ema = ema_ref[...] * EMA_DECAY + (1.0 - EMA_DECAY) * e_mean
    ent = -jnp.sum(new_ema * jnp.log(new_ema + 1e-10))
    m = jnp.sum(mind_ref[...]) / jnp.float32(N * D)
    loss = BETA * m + m
    sub = lax.broadcasted_iota(jnp.int32, (8, 128), 0)
    lane = lax.broadcasted_iota(jnp.int32, (8, 128), 1)
    r0 = sub == 0
    out_ref[...] = jnp.where(
        r0 & (lane == 0), loss,
        jnp.where(r0 & (lane == 1), perp,
                  jnp.where(r0 & (lane == 2), ent, 0.0))).astype(jnp.float32)


def _finalize(mind, pcounts, ema1):
    return pl.pallas_call(
        _k2_body,
        out_shape=jax.ShapeDtypeStruct((8, 128), jnp.float32),
    )(mind, pcounts.reshape(NW, K), ema1)


@functools.partial(jax.jit)
def kernel(z, embedding_weight, proj_W, proj_b, ema_usage):
    z2d = z.reshape(N, D)
    b2d = proj_b.reshape(1, D)
    qc, qct, cnb = _project(embedding_weight, proj_W, b2d)
    idx, mind = _argmin_dist(z2d, qct, cnb)
    ones = jnp.ones((GW,), jnp.float32)
    zeros = jnp.zeros((K,), jnp.float32)
    zq, hists = _sc_gather_hist(qc, idx.reshape(N), ones, zeros)
    out2 = _finalize(mind, hists, ema_usage.reshape(1, K))
    z_q_st = zq.reshape(z.shape)
    return (z_q_st, out2[0, 0], out2[0, 1], out2[0, 2])
